# Initial kernel scaffold; baseline (speedup 1.0000x reference)
import jax
import jax.numpy as jnp
from jax.experimental import pallas as pl


def kernel(cls_target, ctr_target, reg_target, centers, score_threshold, iou_threshold):
    raise NotImplementedError("write your pallas kernel here")


# speedup vs baseline: 16.6501x; 16.6501x over previous
import functools

import jax
import jax.numpy as jnp
from jax.experimental import pallas as pl
from jax.experimental.pallas import tpu as pltpu

H = 100
W = 200
N = H * W
NUM_CLASSES = 80
MAX_OUT = 300
NP = 20480
R = NP // 128
NEG_INF = float("-inf")


def _nms_body(cls_ref, ctr_ref, reg_ref, cen_ref, thr_ref, iou_thr_ref, out_ref):
    thr = thr_ref[0, 0]
    iou_thr = iou_thr_ref[0, 0]

    def cls_step(c, carry):
        acc, amax = carry
        x = cls_ref[c]
        gt = x > acc
        acc = jnp.where(gt, x, acc)
        amax = jnp.where(gt, c, amax)
        return acc, amax

    acc0 = cls_ref[0]
    amax0 = jnp.zeros((R, 128), jnp.int32)
    cls_scores, cls_ids = jax.lax.fori_loop(1, NUM_CLASSES, cls_step, (acc0, amax0))

    score = cls_scores * ctr_ref[...]

    fi = (jax.lax.broadcasted_iota(jnp.int32, (R, 128), 0) * 128
          + jax.lax.broadcasted_iota(jnp.int32, (R, 128), 1))
    valid_loc = fi < N

    s0 = jnp.where((score > thr) & valid_loc, score, NEG_INF)

    cx = cen_ref[0]
    cy = cen_ref[1]
    x1 = cx - reg_ref[0]
    y1 = cy - reg_ref[1]
    x2 = cx + reg_ref[2]
    y2 = cy + reg_ref[3]
    area = (x2 - x1) * (y2 - y1)

    li = jax.lax.broadcasted_iota(jnp.int32, (1, 128), 1)

    def nms_step(k, s):
        m = jnp.max(s)
        idx = jnp.min(jnp.where(s == m, fi, jnp.int32(2**30)))
        valid = m > NEG_INF
        sel = fi == idx
        wx1 = jnp.max(jnp.where(sel, x1, NEG_INF))
        wy1 = jnp.max(jnp.where(sel, y1, NEG_INF))
        wx2 = jnp.max(jnp.where(sel, x2, NEG_INF))
        wy2 = jnp.max(jnp.where(sel, y2, NEG_INF))
        wid = jnp.max(jnp.where(sel, cls_ids, -1))

        ix1 = jnp.maximum(wx1, x1)
        iy1 = jnp.maximum(wy1, y1)
        ix2 = jnp.minimum(wx2, x2)
        iy2 = jnp.minimum(wy2, y2)
        inter = jnp.maximum(ix2 - ix1, 0.0) * jnp.maximum(iy2 - iy1, 0.0)
        warea = (wx2 - wx1) * (wy2 - wy1)
        iou = inter / (warea + area - inter + 1e-8)
        s = jnp.where((iou > iou_thr) | sel, NEG_INF, s)

        vf = valid.astype(jnp.float32)
        row = jnp.where(li == 0, wx1 * vf,
              jnp.where(li == 1, wy1 * vf,
              jnp.where(li == 2, wx2 * vf,
              jnp.where(li == 3, wy2 * vf,
              jnp.where(li == 4, m * vf,
              jnp.where(li == 5,
                        jnp.where(valid, wid, -1).astype(jnp.float32),
                        0.0))))))
        out_ref[pl.ds(k, 1), :] = row
        return s

    jax.lax.fori_loop(0, MAX_OUT, nms_step, s0, unroll=False)


@jax.jit
def _decode_nms(cls_t, ctr_t, reg_t, centers, score_threshold, iou_threshold):
    pad = NP - N
    cls_p = jnp.pad(cls_t[0].T, ((0, 0), (0, pad))).reshape(NUM_CLASSES, R, 128)
    ctr_p = jnp.pad(ctr_t[0], ((0, pad),)).reshape(R, 128)
    reg_p = jnp.pad(reg_t[0].T, ((0, 0), (0, pad))).reshape(4, R, 128)
    cen_p = jnp.pad(centers.T, ((0, 0), (0, pad))).reshape(2, R, 128)
    thr = jnp.asarray(score_threshold, jnp.float32).reshape(1, 1)
    iou_thr = jnp.asarray(iou_threshold, jnp.float32).reshape(1, 1)

    out = pl.pallas_call(
        _nms_body,
        out_shape=jax.ShapeDtypeStruct((304, 128), jnp.float32),
        in_specs=[
            pl.BlockSpec(memory_space=pltpu.VMEM),
            pl.BlockSpec(memory_space=pltpu.VMEM),
            pl.BlockSpec(memory_space=pltpu.VMEM),
            pl.BlockSpec(memory_space=pltpu.VMEM),
            pl.BlockSpec(memory_space=pltpu.SMEM),
            pl.BlockSpec(memory_space=pltpu.SMEM),
        ],
        out_specs=pl.BlockSpec(memory_space=pltpu.VMEM),
    )(cls_p, ctr_p, reg_p, cen_p, thr, iou_thr)

    sel = out[:MAX_OUT]
    out_boxes = sel[:, 0:4][None]
    out_scores = sel[:, 4][None]
    out_ids = sel[:, 5].astype(jnp.int32)[None]
    return out_boxes, out_scores, out_ids


def kernel(cls_target, ctr_target, reg_target, centers, score_threshold, iou_threshold):
    return _decode_nms(cls_target, ctr_target, reg_target, centers,
                       score_threshold, iou_threshold)
